# Initial kernel scaffold; baseline (speedup 1.0000x reference)
import jax
import jax.numpy as jnp
from jax.experimental import pallas as pl


def kernel(loss):
    raise NotImplementedError("write your pallas kernel here")


# speedup vs baseline: 18.1231x; 18.1231x over previous
import functools

import jax
import jax.numpy as jnp
from jax import lax
from jax.experimental import pallas as pl
from jax.experimental.pallas import tpu as pltpu
from jax.experimental.pallas import tpu_sc as plsc

N = 4194304
NC, NS, L = 2, 16, 16
NW = NC * NS
PER_W = N // NW
WIN = 8192
NWIN = PER_W // WIN
VPW = WIN // L
UNROLL = 4

K1 = 209716
K2 = 3984589
CNT = K2 - K1

_mesh = plsc.VectorSubcoreMesh(
    core_axis_name="c", subcore_axis_name="s", num_cores=NC, num_subcores=NS)

_f32 = jnp.float32
_i32 = jnp.int32
_u32 = jnp.uint32


def _c32(val):
    return jnp.full((L,), val, _u32)


def _ci(val):
    return jnp.full((L,), val, _i32)


def _key_of(x):
    u = plsc.bitcast(x, _u32)
    neg = u >= _c32(0x80000000)
    return u ^ jnp.where(neg, _c32(0xFFFFFFFF), _c32(0x80000000))


def _wid():
    return lax.axis_index("s") * NC + lax.axis_index("c")


def _zero_words(ref, nwords):
    z = jnp.zeros((L,), _i32)

    def body(i, _):
        ref[pl.ds(i * L, L)] = z
        return 0

    lax.fori_loop(0, nwords // L, body, 0)


def _reduce_rows(h_hbm, rowbuf, acc, nrows, nwords):
    _zero_words(acc, nwords)

    def row(w, _):
        pltpu.sync_copy(h_hbm.at[w], rowbuf)

        def add(i, _):
            acc[pl.ds(i * L, L)] = acc[pl.ds(i * L, L)] + rowbuf[pl.ds(i * L, L)]
            return 0

        lax.fori_loop(0, nwords // L, add, 0)
        return 0

    lax.fori_loop(0, nrows, row, 0)


def _select(acc, base, nbuckets, kr):
    krv = _ci(kr)
    zi = jnp.zeros((L,), _i32)

    def chunk(i, carry):
        tot, pcnt, cbel = carry
        v = acc[pl.ds(base + i * L, L)]
        cum = plsc.cumsum(v) + _ci(tot)
        lt = cum < krv
        pcnt = pcnt + jnp.where(lt, _ci(1), zi)
        cbel = cbel + jnp.where(lt, v, zi)
        return jnp.max(cum), pcnt, cbel

    tot, pcnt, cbel = lax.fori_loop(
        0, nbuckets // L, chunk, (jnp.int32(0), zi, zi))
    del tot
    return jnp.sum(pcnt), jnp.sum(cbel)


def _lane_reduce(tbl, hloc, slots, nbuckets):
    for t in range(slots):

        def body(cch, _, t=t):
            acc = jnp.zeros((L,), _i32)
            for l in range(L):
                acc = acc + tbl[pl.ds((t * L + l) * nbuckets + cch * L, L)]
            hloc[pl.ds(t * nbuckets + cch * L, L)] = acc
            return 0

        lax.fori_loop(0, nbuckets // L, body, 0)


@functools.partial(
    pl.kernel,
    out_type=jax.ShapeDtypeStruct((NW, 2048), _i32),
    mesh=_mesh,
    compiler_params=pltpu.CompilerParams(needs_layout_passes=False),
    scratch_types=[
        pltpu.VMEM((WIN,), _f32),
        pltpu.VMEM((L * 2048,), _i32),
        pltpu.VMEM((2048,), _i32),
    ],
)
def _ka(loss, hout, buf, tbl, hloc):
    wid = _wid()
    base = wid * PER_W
    lanebase = lax.iota(_i32, L) * 2048
    ones = jnp.ones((L,), _i32)
    _zero_words(tbl, L * 2048)

    def win(w, _):
        pltpu.sync_copy(loss.at[pl.ds(base + w * WIN, WIN)], buf)

        def step(i, _):
            for j in range(UNROLL):
                x = buf[pl.ds((i * UNROLL + j) * L, L)]
                b = plsc.bitcast(_key_of(x) >> _c32(21), _i32)
                plsc.addupdate_scatter(tbl, [lanebase + b], ones)
            return 0

        lax.fori_loop(0, VPW // UNROLL, step, 0)
        return 0

    lax.fori_loop(0, NWIN, win, 0)
    _lane_reduce(tbl, hloc, 1, 2048)
    pltpu.sync_copy(hloc, hout.at[wid])


@functools.partial(
    pl.kernel,
    out_type=(
        jax.ShapeDtypeStruct((NW, 4096), _i32),
        jax.ShapeDtypeStruct((L,), _i32),
    ),
    mesh=_mesh,
    compiler_params=pltpu.CompilerParams(needs_layout_passes=False),
    scratch_types=[
        pltpu.VMEM((WIN,), _f32),
        pltpu.VMEM((2 * L * 2048,), _i32),
        pltpu.VMEM((2048,), _i32),
        pltpu.VMEM((2048,), _i32),
        pltpu.VMEM((4096,), _i32),
        pltpu.VMEM((L,), _i32),
    ],
)
def _kb(loss, ha, hout, stout, buf, tbl, rowbuf, acc, hloc, stv):
    wid = _wid()
    base = wid * PER_W
    _reduce_rows(ha, rowbuf, acc, NW, 2048)
    p1, cb1 = _select(acc, 0, 2048, K1)
    p2, cb2 = _select(acc, 0, 2048, K2)
    kr1 = K1 - cb1
    kr2 = K2 - cb2

    pref1 = plsc.bitcast(_ci(p1), _u32)
    pref2 = plsc.bitcast(_ci(p2), _u32)
    lanebase = lax.iota(_i32, L) * 2048
    ones = jnp.ones((L,), _i32)
    _zero_words(tbl, 2 * L * 2048)

    def win(w, _):
        pltpu.sync_copy(loss.at[pl.ds(base + w * WIN, WIN)], buf)

        def step(i, _):
            for j in range(UNROLL):
                x = buf[pl.ds((i * UNROLL + j) * L, L)]
                key = _key_of(x)
                hi = key >> _c32(21)
                b = plsc.bitcast((key >> _c32(10)) & _c32(0x7FF), _i32)
                idx = lanebase + b
                plsc.addupdate_scatter(tbl, [idx], ones, mask=hi == pref1)
                plsc.addupdate_scatter(
                    tbl, [idx + _ci(L * 2048)], ones, mask=hi == pref2)
            return 0

        lax.fori_loop(0, VPW // UNROLL, step, 0)
        return 0

    lax.fori_loop(0, NWIN, win, 0)
    _lane_reduce(tbl, hloc, 2, 2048)
    pltpu.sync_copy(hloc, hout.at[wid])

    iota = lax.iota(_i32, L)
    st = (jnp.where(iota == 0, _ci(p1), _ci(0))
          + jnp.where(iota == 1, _ci(kr1), _ci(0))
          + jnp.where(iota == 2, _ci(p2), _ci(0))
          + jnp.where(iota == 3, _ci(kr2), _ci(0)))
    stv[...] = st

    @pl.when(wid == 0)
    def _():
        pltpu.sync_copy(stv, stout)


@functools.partial(
    pl.kernel,
    out_type=(
        jax.ShapeDtypeStruct((NW, 2048), _i32),
        jax.ShapeDtypeStruct((L,), _i32),
    ),
    mesh=_mesh,
    compiler_params=pltpu.CompilerParams(needs_layout_passes=False),
    scratch_types=[
        pltpu.VMEM((WIN,), _f32),
        pltpu.VMEM((2 * L * 1024,), _i32),
        pltpu.VMEM((4096,), _i32),
        pltpu.VMEM((4096,), _i32),
        pltpu.VMEM((2048,), _i32),
        pltpu.VMEM((L,), _i32),
    ],
)
def _kc(loss, hb, stb, hout, stout, buf, tbl, rowbuf, acc, hloc, stv):
    wid = _wid()
    base = wid * PER_W
    pltpu.sync_copy(stb, stv)
    sv = stv[...]
    p1a = sv[0]
    kr1a = sv[1]
    p2a = sv[2]
    kr2a = sv[3]

    _reduce_rows(hb, rowbuf, acc, NW, 4096)
    p1b, cb1 = _select(acc, 0, 2048, kr1a)
    p2b, cb2 = _select(acc, 2048, 2048, kr2a)
    kr1 = kr1a - cb1
    kr2 = kr2a - cb2
    pref1 = ((plsc.bitcast(_ci(p1a), _u32) << _c32(11))
             | plsc.bitcast(_ci(p1b), _u32))
    pref2 = ((plsc.bitcast(_ci(p2a), _u32) << _c32(11))
             | plsc.bitcast(_ci(p2b), _u32))

    lanebase = lax.iota(_i32, L) * 1024
    ones = jnp.ones((L,), _i32)
    _zero_words(tbl, 2 * L * 1024)

    def win(w, _):
        pltpu.sync_copy(loss.at[pl.ds(base + w * WIN, WIN)], buf)

        def step(i, _):
            for j in range(UNROLL):
                x = buf[pl.ds((i * UNROLL + j) * L, L)]
                key = _key_of(x)
                hi = key >> _c32(10)
                b = plsc.bitcast(key & _c32(0x3FF), _i32)
                idx = lanebase + b
                plsc.addupdate_scatter(tbl, [idx], ones, mask=hi == pref1)
                plsc.addupdate_scatter(
                    tbl, [idx + _ci(L * 1024)], ones, mask=hi == pref2)
            return 0

        lax.fori_loop(0, VPW // UNROLL, step, 0)
        return 0

    lax.fori_loop(0, NWIN, win, 0)
    _lane_reduce(tbl, hloc, 2, 1024)
    pltpu.sync_copy(hloc, hout.at[wid])

    pr1 = (p1a << 11) | p1b
    pr2 = (p2a << 11) | p2b
    iota = lax.iota(_i32, L)
    st = (jnp.where(iota == 0, _ci(pr1), _ci(0))
          + jnp.where(iota == 1, _ci(kr1), _ci(0))
          + jnp.where(iota == 2, _ci(pr2), _ci(0))
          + jnp.where(iota == 3, _ci(kr2), _ci(0)))
    stv[...] = st

    @pl.when(wid == 0)
    def _():
        pltpu.sync_copy(stv, stout)


@functools.partial(
    pl.kernel,
    out_type=jax.ShapeDtypeStruct((NW, L), _f32),
    mesh=_mesh,
    compiler_params=pltpu.CompilerParams(needs_layout_passes=False),
    scratch_types=[
        pltpu.VMEM((WIN,), _f32),
        pltpu.VMEM((2048,), _i32),
        pltpu.VMEM((2048,), _i32),
        pltpu.VMEM((L,), _i32),
        pltpu.VMEM((L,), _f32),
    ],
)
def _kd(loss, hc, stc, sout, buf, rowbuf, acc, stv, srow):
    wid = _wid()
    base = wid * PER_W
    pltpu.sync_copy(stc, stv)
    sv = stv[...]
    pr1 = sv[0]
    kr1a = sv[1]
    pr2 = sv[2]
    kr2a = sv[3]

    _reduce_rows(hc, rowbuf, acc, NW, 2048)
    p1c, cb1 = _select(acc, 0, 1024, kr1a)
    p2c, cb2 = _select(acc, 1024, 1024, kr2a)
    kr1 = kr1a - cb1
    kr2 = kr2a - cb2
    t1 = ((plsc.bitcast(_ci(pr1), _u32) << _c32(10))
          | plsc.bitcast(_ci(p1c), _u32))
    t2 = ((plsc.bitcast(_ci(pr2), _u32) << _c32(10))
          | plsc.bitcast(_ci(p2c), _u32))

    zf = jnp.zeros((L,), _f32)

    def win(w, carry):
        pltpu.sync_copy(loss.at[pl.ds(base + w * WIN, WIN)], buf)

        def step(i, c):
            a1, a2 = c
            for j in range(UNROLL):
                x = buf[pl.ds((i * UNROLL + j) * L, L)]
                key = _key_of(x)
                a1 = a1 + jnp.where(key < t1, x, zf)
                a2 = a2 + jnp.where(key < t2, x, zf)
            return a1, a2

        return lax.fori_loop(0, VPW // UNROLL, step, carry)

    s1, s2 = lax.fori_loop(0, NWIN, win, (zf, zf))

    def decode(t):
        pos = t >= _c32(0x80000000)
        bits = t ^ jnp.where(pos, _c32(0x80000000), _c32(0xFFFFFFFF))
        return plsc.bitcast(bits, _f32)

    iota = lax.iota(_i32, L)
    lane0 = jnp.logical_and(iota == 0, _ci(wid) == 0)
    corr1 = jnp.where(lane0, _ci(kr1).astype(_f32) * decode(t1), zf)
    corr2 = jnp.where(lane0, _ci(kr2).astype(_f32) * decode(t2), zf)
    srow[...] = (s2 + corr2) - (s1 + corr1)
    pltpu.sync_copy(srow, sout.at[wid])


def _ke_body(sp_ref, o_ref):
    t = jnp.sum(sp_ref[...]) / jnp.float32(CNT)
    o_ref[...] = jnp.full((8, 128), t, _f32)


_ke = pl.pallas_call(
    _ke_body,
    out_shape=jax.ShapeDtypeStruct((8, 128), _f32),
)


def kernel(loss):
    assert loss.shape == (N,) and loss.dtype == jnp.float32
    ha = _ka(loss)
    hb, stb = _kb(loss, ha)
    hc, stc = _kc(loss, hb, stb)
    sp = _kd(loss, hc, stc)
    out = _ke(sp)
    return out[0, 0]


# speedup vs baseline: 37.6764x; 2.0789x over previous
import functools

import jax
import jax.numpy as jnp
from jax import lax
from jax.experimental import pallas as pl
from jax.experimental.pallas import tpu as pltpu
from jax.experimental.pallas import tpu_sc as plsc

N = 4194304
NC, NS, L = 2, 16, 16
NW = NC * NS
PER_W = N // NW
WIN = 8192
NWIN = PER_W // WIN
VPW = WIN // L

CAP = 8192
CAPR = CAP + 32
NVROW = CAPR // L

K1 = 209716
K2 = 3984589
CNT = K2 - K1

_mesh = plsc.VectorSubcoreMesh(
    core_axis_name="c", subcore_axis_name="s", num_cores=NC, num_subcores=NS)
_params = pltpu.CompilerParams(needs_layout_passes=False)

_f32 = jnp.float32
_i32 = jnp.int32
_u32 = jnp.uint32


def _c32(val):
    return jnp.full((L,), val, _u32)


def _ci(val):
    return jnp.full((L,), val, _i32)


def _key_of(x):
    u = plsc.bitcast(x, _u32)
    neg = u >= _c32(0x80000000)
    return u ^ jnp.where(neg, _c32(0xFFFFFFFF), _c32(0x80000000))


def _wid():
    return lax.axis_index("s") * NC + lax.axis_index("c")


def _zero_words(ref, nwords, dtype=_i32):
    z = jnp.zeros((L,), dtype)

    @plsc.parallel_loop(0, nwords // L, unroll=8)
    def _(i):
        ref[pl.ds(i * L, L)] = z


def _zero2d(tbl, nb, dtype=_i32):
    z = jnp.zeros((L,), dtype)
    for l in range(L):

        @plsc.parallel_loop(0, nb // L, unroll=8)
        def _(i, l=l):
            tbl[l, pl.ds(i * L, L)] = z


def _sum16rows(tbl, dst, nb, dtype=_i32):
    @plsc.parallel_loop(0, nb // L, unroll=2)
    def _(c):
        a = jnp.zeros((L,), dtype)
        for r in range(L):
            a = a + tbl[r, pl.ds(c * L, L)]
        dst[pl.ds(c * L, L)] = a


def _reduce_rows(h_hbm, blockbuf, acc, nrows, nwords, rpb, dtype=_i32):
    _zero_words(acc, nwords, dtype)

    def blk(bi, _):
        pltpu.sync_copy(h_hbm.at[pl.ds(bi * rpb, rpb)], blockbuf)

        @plsc.parallel_loop(0, nwords // L, unroll=2)
        def _(i):
            a = acc[pl.ds(i * L, L)]
            for r in range(rpb):
                a = a + blockbuf[r, pl.ds(i * L, L)]
            acc[pl.ds(i * L, L)] = a

        return 0

    lax.fori_loop(0, nrows // rpb, blk, 0)


def _select(acc, base, nbuckets, kr, sacc=None):
    krv = _ci(kr)
    zi = jnp.zeros((L,), _i32)
    zf = jnp.zeros((L,), _f32)

    def chunk(i, carry):
        tot, pcnt, cbel, sbel = carry
        v = acc[pl.ds(base + i * L, L)]
        cum = plsc.cumsum(v) + _ci(tot)
        lt = cum < krv
        pcnt = pcnt + jnp.where(lt, _ci(1), zi)
        cbel = cbel + jnp.where(lt, v, zi)
        if sacc is not None:
            sv = sacc[pl.ds(base + i * L, L)]
            sbel = sbel + jnp.where(lt, sv, zf)
        return jnp.max(cum), pcnt, cbel, sbel

    tot, pcnt, cbel, sbel = lax.fori_loop(
        0, nbuckets // L, chunk, (jnp.int32(0), zi, zi, zf))
    del tot
    if sacc is None:
        return jnp.sum(pcnt), jnp.sum(cbel)
    return jnp.sum(pcnt), jnp.sum(cbel), sbel


def _lane_reduce(tbl, hloc, slots, nbuckets, dtype=_i32):
    for t in range(slots):

        @plsc.parallel_loop(0, nbuckets // L, unroll=4)
        def _(cch, t=t):
            acc = jnp.zeros((L,), dtype)
            for l in range(L):
                acc = acc + tbl[pl.ds((t * L + l) * nbuckets + cch * L, L)]
            hloc[pl.ds(t * nbuckets + cch * L, L)] = acc


def _stream(loss, buf2, sems, base, compute, carry):
    def _start(w, par):
        pltpu.async_copy(
            loss.at[pl.ds(base + w * WIN, WIN)], buf2.at[par], sems[par])

    def _wait(w, par):
        pltpu.make_async_copy(
            loss.at[pl.ds(base + w * WIN, WIN)], buf2.at[par], sems[par]).wait()

    _start(0, 0)
    _start(1, 1)

    def pair(w2, c):
        w = w2 * 2
        for par in range(2):

            _wait(w + par, par)
            c = compute(par, c)

            @pl.when(w + par + 2 < NWIN)
            def _():
                _start(w + par + 2, par)

        return c

    return lax.fori_loop(0, NWIN // 2, pair, carry)


def _bcast_state(stv, sh_st, a, b):
    iota = lax.iota(_i32, L)
    stv[...] = (jnp.where(iota == 0, _ci(a), _ci(0))
                + jnp.where(iota == 1, _ci(b), _ci(0)))
    pltpu.sync_copy(stv, sh_st)


def _read_state(stv, sh_st):
    pltpu.sync_copy(sh_st, stv)
    sv = stv[...]
    return sv[0], sv[1]


@functools.partial(
    pl.kernel,
    out_type=(
        jax.ShapeDtypeStruct((NW, 2048), _i32),
        jax.ShapeDtypeStruct((NW, 2048), _f32),
    ),
    mesh=_mesh,
    compiler_params=_params,
    scratch_types=[
        pltpu.VMEM((2, WIN), _f32),
        pltpu.SemaphoreType.DMA,
        pltpu.SemaphoreType.DMA,
        pltpu.VMEM((L * 2048,), _i32),
        pltpu.VMEM((L * 2048,), _f32),
        pltpu.VMEM((2048,), _i32),
        pltpu.VMEM((2048,), _f32),
    ],
)
def _ka(loss, hout, sout, buf2, sem0, sem1, tblc, tbls, hloc, sloc):
    wid = _wid()
    base = wid * PER_W
    laneb = plsc.bitcast(lax.iota(_i32, L) * 2048, _u32)
    ones = jnp.ones((L,), _i32)
    _zero_words(tblc, L * 2048)
    _zero_words(tbls, L * 2048, _f32)

    def compute(par, c):

        @plsc.parallel_loop(0, VPW, unroll=8)
        def _(i):
            x = buf2[par, pl.ds(i * L, L)]
            idx = plsc.bitcast(laneb | (_key_of(x) >> _c32(21)), _i32)
            plsc.addupdate_scatter(tblc, [idx], ones)
            plsc.addupdate_scatter(tbls, [idx], x)

        return c

    _stream(loss, buf2, (sem0, sem1), base, compute, 0)
    _lane_reduce(tblc, hloc, 1, 2048)
    _lane_reduce(tbls, sloc, 1, 2048, _f32)
    pltpu.sync_copy(hloc, hout.at[wid])
    pltpu.sync_copy(sloc, sout.at[wid])


@functools.partial(
    pl.kernel,
    out_type=jax.ShapeDtypeStruct((2, NW, CAPR), _f32),
    mesh=_mesh,
    compiler_params=_params,
    scratch_types=[
        pltpu.VMEM((2, WIN), _f32),
        pltpu.SemaphoreType.DMA,
        pltpu.SemaphoreType.DMA,
        pltpu.VMEM((8, 2048), _i32),
        pltpu.VMEM((2048,), _i32),
        pltpu.VMEM((CAPR,), _f32),
        pltpu.VMEM((CAPR,), _f32),
    ],
)
def _kb(loss, ha, cm, buf2, sem0, sem1, blockbuf, acc, cbuf1, cbuf2):
    wid = _wid()
    base = wid * PER_W
    _reduce_rows(ha, blockbuf, acc, NW, 2048, 8)
    p1, _cb1 = _select(acc, 0, 2048, K1)
    p2, _cb2 = _select(acc, 0, 2048, K2)
    pref1 = plsc.bitcast(_ci(p1), _u32)
    pref2 = plsc.bitcast(_ci(p2), _u32)

    def compute(par, c):

        @plsc.parallel_loop(0, VPW, unroll=8, carry=c)
        def inner(i, cc):
            o1, o2 = cc
            x = buf2[par, pl.ds(i * L, L)]
            hi = _key_of(x) >> _c32(21)
            m1 = jnp.logical_and(hi == pref1, _ci(o1) < _ci(CAP))
            m2 = jnp.logical_and(hi == pref2, _ci(o2) < _ci(CAP))
            plsc.store_compressed(cbuf1.at[pl.ds(o1, L)], x, mask=m1)
            plsc.store_compressed(cbuf2.at[pl.ds(o2, L)], x, mask=m2)
            n1 = plsc.all_reduce_population_count(m1)
            n2 = plsc.all_reduce_population_count(m2)
            o1 = o1 + (n1[0] if n1.ndim else n1)
            o2 = o2 + (n2[0] if n2.ndim else n2)
            return o1, o2

        return inner

    o1, o2 = _stream(loss, buf2, (sem0, sem1), base, compute,
                     (jnp.int32(0), jnp.int32(0)))

    nanv = plsc.bitcast(_c32(0x7FC00000), _f32)
    for cbuf, off in ((cbuf1, o1), (cbuf2, o2)):
        cbuf[pl.ds(off, L)] = nanv
        v0 = (off >> 4) + 1

        def fill(v, _, cbuf=cbuf):
            cbuf[pl.ds(v * L, L)] = nanv
            return 0

        lax.fori_loop(v0, CAPR // L, fill, 0)

    pltpu.sync_copy(cbuf1, cm.at[0, wid])
    pltpu.sync_copy(cbuf2, cm.at[1, wid])


@functools.partial(
    pl.kernel,
    out_type=jax.ShapeDtypeStruct((2, L), _f32),
    mesh=_mesh,
    compiler_params=_params,
    scratch_types=[
        pltpu.VMEM((4, 2048), _i32),
        pltpu.VMEM((4, 2048), _f32),
        pltpu.VMEM((2048,), _i32),
        pltpu.VMEM((2048,), _f32),
        pltpu.VMEM((2, CAPR), _f32),
        pltpu.VMEM((L, 2048), _i32),
        pltpu.VMEM((2048,), _i32),
        pltpu.VMEM((L,), _i32),
        pltpu.VMEM((L,), _f32),
        pltpu.VMEM((L,), _f32),
        pltpu.VMEM_SHARED((L, 2048), _i32),
        pltpu.VMEM_SHARED((L,), _i32),
    ],
)
def _kf(ha, sa, cm, rout,
        bbc, bbs, acc, sacc, data, tbl, hloc, stv, sbstash, wrow,
        sh_c, sh_st):
    cid = lax.axis_index("c")
    sid = lax.axis_index("s")
    ktarget = jnp.where(cid == 0, K1, K2)
    iota = lax.iota(_i32, L)
    ones = jnp.ones((L,), _i32)
    zf = jnp.zeros((L,), _f32)

    @pl.when(sid == 0)
    def _():
        _reduce_rows(ha, bbc, acc, NW, 2048, 4)
        _reduce_rows(sa, bbs, sacc, NW, 2048, 4, _f32)
        p, cb, sb = _select(acc, 0, 2048, ktarget, sacc)
        sbstash[...] = sb
        _bcast_state(stv, sh_st, p, ktarget - cb)

    plsc.subcore_barrier()
    p_l1, kr_l1 = _read_state(stv, sh_st)
    prefv1 = plsc.bitcast(_ci(p_l1), _u32)

    pltpu.sync_copy(cm.at[cid, pl.ds(sid * 2, 2)], data)

    _zero2d(tbl, 2048)
    for r in range(2):

        @plsc.parallel_loop(0, NVROW, unroll=8)
        def _(v, r=r):
            x = data[r, pl.ds(v * L, L)]
            key = _key_of(x)
            m = (key >> _c32(21)) == prefv1
            b = plsc.bitcast((key >> _c32(10)) & _c32(0x7FF), _i32)
            plsc.addupdate_scatter(tbl, [iota, b], ones, mask=m)

    @plsc.parallel_loop(0, 2048 // L, unroll=4)
    def _(c):
        a = jnp.zeros((L,), _i32)
        for l in range(L):
            a = a + tbl[l, pl.ds(c * L, L)]
        hloc[pl.ds(c * L, L)] = a

    pltpu.sync_copy(hloc, sh_c.at[sid])
    plsc.subcore_barrier()

    @pl.when(sid == 0)
    def _():
        pltpu.sync_copy(sh_c, tbl)
        _sum16rows(tbl, acc, 2048)
        pb, cbb = _select(acc, 0, 2048, kr_l1)
        _bcast_state(stv, sh_st, pb, kr_l1 - cbb)

    plsc.subcore_barrier()
    p_l2, kr_l2 = _read_state(stv, sh_st)
    pref22 = (prefv1 << _c32(11)) | plsc.bitcast(_ci(p_l2), _u32)

    _zero2d(tbl, 1024)
    for r in range(2):

        @plsc.parallel_loop(0, NVROW, unroll=8)
        def _(v, r=r):
            x = data[r, pl.ds(v * L, L)]
            key = _key_of(x)
            m = (key >> _c32(10)) == pref22
            b = plsc.bitcast(key & _c32(0x3FF), _i32)
            plsc.addupdate_scatter(tbl, [iota, b], ones, mask=m)

    @plsc.parallel_loop(0, 1024 // L, unroll=4)
    def _(c):
        a = jnp.zeros((L,), _i32)
        for l in range(L):
            a = a + tbl[l, pl.ds(c * L, L)]
        hloc[pl.ds(c * L, L)] = a

    pltpu.sync_copy(hloc.at[pl.ds(0, 1024)], sh_c.at[sid, pl.ds(0, 1024)])
    plsc.subcore_barrier()

    @pl.when(sid == 0)
    def _():
        pltpu.sync_copy(sh_c, tbl)
        _sum16rows(tbl, acc, 1024)
        pc, cbc = _select(acc, 0, 1024, kr_l2)
        _bcast_state(stv, sh_st, pc, kr_l2 - cbc)

    plsc.subcore_barrier()
    p_l3, kr_f = _read_state(stv, sh_st)
    tv = (pref22 << _c32(10)) | plsc.bitcast(_ci(p_l3), _u32)

    w = zf
    for r in range(2):

        @plsc.parallel_loop(0, NVROW, unroll=8, carry=w)
        def w(v, c, r=r):
            x = data[r, pl.ds(v * L, L)]
            return c + jnp.where(_key_of(x) < tv, x, zf)

    hloc[pl.ds(0, L)] = plsc.bitcast(w, _i32)
    pltpu.sync_copy(hloc, sh_c.at[sid])
    plsc.subcore_barrier()

    @pl.when(sid == 0)
    def _():
        pltpu.sync_copy(sh_c, tbl)
        wtot = zf
        for r in range(L):
            wtot = wtot + plsc.bitcast(tbl[r, pl.ds(0, L)], _f32)

        def decode(t):
            pos = t >= _c32(0x80000000)
            bits = t ^ jnp.where(pos, _c32(0x80000000), _c32(0xFFFFFFFF))
            return plsc.bitcast(bits, _f32)

        corr = jnp.where(iota == 0, _ci(kr_f).astype(_f32) * decode(tv), zf)
        wrow[...] = sbstash[...] + wtot + corr
        pltpu.sync_copy(wrow, rout.at[cid])


def _ke_body(sp_ref, o_ref):
    sp = sp_ref[...]
    t = (jnp.sum(sp[1, :]) - jnp.sum(sp[0, :])) / jnp.float32(CNT)
    o_ref[...] = jnp.full((8, 128), t, _f32)


_ke = pl.pallas_call(
    _ke_body,
    out_shape=jax.ShapeDtypeStruct((8, 128), _f32),
)


def kernel(loss):
    assert loss.shape == (N,) and loss.dtype == jnp.float32
    ha, sa = _ka(loss)
    cm = _kb(loss, ha)
    sb = _kf(ha, sa, cm)
    out = _ke(sb)
    return out[0, 0]


# speedup vs baseline: 51.8680x; 1.3767x over previous
import functools

import jax
import jax.numpy as jnp
from jax import lax
from jax.experimental import pallas as pl
from jax.experimental.pallas import tpu as pltpu
from jax.experimental.pallas import tpu_sc as plsc

N = 4194304
NC, NS, L = 2, 16, 16
NW = NC * NS
PER_W = N // NW
WIN = 8192
NWIN = PER_W // WIN
VPW = WIN // L

K1 = 209716
K2 = 3984589
CNT = K2 - K1

_mesh = plsc.VectorSubcoreMesh(
    core_axis_name="c", subcore_axis_name="s", num_cores=NC, num_subcores=NS)
_params = pltpu.CompilerParams(needs_layout_passes=False)

_f32 = jnp.float32
_i32 = jnp.int32
_u32 = jnp.uint32


def _c32(val):
    return jnp.full((L,), val, _u32)


def _ci(val):
    return jnp.full((L,), val, _i32)


def _cf(val):
    return jnp.full((L,), val, _f32)


def _key_of(x):
    u = plsc.bitcast(x, _u32)
    neg = u >= _c32(0x80000000)
    return u ^ jnp.where(neg, _c32(0xFFFFFFFF), _c32(0x80000000))


def _wid():
    return lax.axis_index("s") * NC + lax.axis_index("c")


def _zero_words(ref, nwords, dtype=_i32):
    z = jnp.zeros((L,), dtype)

    @plsc.parallel_loop(0, nwords // L, unroll=8)
    def _(i):
        ref[pl.ds(i * L, L)] = z


def _reduce_rows(h_hbm, blockbuf, acc, nrows, nwords, rpb, dtype=_i32):
    _zero_words(acc, nwords, dtype)

    def blk(bi, _):
        pltpu.sync_copy(h_hbm.at[pl.ds(bi * rpb, rpb)], blockbuf)

        @plsc.parallel_loop(0, nwords // L, unroll=2)
        def _(i):
            a = acc[pl.ds(i * L, L)]
            for r in range(rpb):
                a = a + blockbuf[r, pl.ds(i * L, L)]
            acc[pl.ds(i * L, L)] = a

        return 0

    lax.fori_loop(0, nrows // rpb, blk, 0)


def _stream(loss, buf2, sems, base, compute, carry):
    def _start(w, par):
        pltpu.async_copy(
            loss.at[pl.ds(base + w * WIN, WIN)], buf2.at[par], sems[par])

    def _wait(w, par):
        pltpu.make_async_copy(
            loss.at[pl.ds(base + w * WIN, WIN)], buf2.at[par], sems[par]).wait()

    _start(0, 0)
    _start(1, 1)

    def pair(w2, c):
        w = w2 * 2
        for par in range(2):

            _wait(w + par, par)
            c = compute(par, c)

            @pl.when(w + par + 2 < NWIN)
            def _():
                _start(w + par + 2, par)

        return c

    return lax.fori_loop(0, NWIN // 2, pair, carry)


def _select(acc, base, nbuckets, kr, sacc=None):
    krv = _ci(kr)
    zi = jnp.zeros((L,), _i32)
    zf = jnp.zeros((L,), _f32)

    def chunk(i, carry):
        tot, pcnt, cbel, sbel = carry
        v = acc[pl.ds(base + i * L, L)]
        cum = plsc.cumsum(v) + _ci(tot)
        lt = cum < krv
        pcnt = pcnt + jnp.where(lt, _ci(1), zi)
        cbel = cbel + jnp.where(lt, v, zi)
        if sacc is not None:
            sv = sacc[pl.ds(base + i * L, L)]
            sbel = sbel + jnp.where(lt, sv, zf)
        return jnp.max(cum), pcnt, cbel, sbel

    tot, pcnt, cbel, sbel = lax.fori_loop(
        0, nbuckets // L, chunk, (jnp.int32(0), zi, zi, zf))
    del tot
    if sacc is None:
        return jnp.sum(pcnt), jnp.sum(cbel)
    return jnp.sum(pcnt), jnp.sum(cbel), sbel


def _lane_reduce(tbl, hloc, slots, nbuckets, dtype=_i32):
    for t in range(slots):

        @plsc.parallel_loop(0, nbuckets // L, unroll=4)
        def _(cch, t=t):
            acc = jnp.zeros((L,), dtype)
            for l in range(L):
                acc = acc + tbl[pl.ds((t * L + l) * nbuckets + cch * L, L)]
            hloc[pl.ds(t * nbuckets + cch * L, L)] = acc


@functools.partial(
    pl.kernel,
    out_type=jax.ShapeDtypeStruct((NW, 2048), _i32),
    mesh=_mesh,
    compiler_params=_params,
    scratch_types=[
        pltpu.VMEM((2, WIN), _f32),
        pltpu.SemaphoreType.DMA,
        pltpu.SemaphoreType.DMA,
        pltpu.VMEM((L * 2048,), _i32),
        pltpu.VMEM((2048,), _i32),
    ],
)
def _ka(loss, hout, buf2, sem0, sem1, tbl, hloc):
    wid = _wid()
    base = wid * PER_W
    laneb = plsc.bitcast(lax.iota(_i32, L) * 2048, _u32)
    ones = jnp.ones((L,), _i32)
    _zero_words(tbl, L * 2048)

    def compute(par, c):

        @plsc.parallel_loop(0, VPW, unroll=16)
        def _(i):
            x = buf2[par, pl.ds(i * L, L)]
            idx = plsc.bitcast(laneb | (_key_of(x) >> _c32(21)), _i32)
            plsc.addupdate_scatter(tbl, [idx], ones)

        return c

    _stream(loss, buf2, (sem0, sem1), base, compute, 0)
    _lane_reduce(tbl, hloc, 1, 2048)
    pltpu.sync_copy(hloc, hout.at[wid])


@functools.partial(
    pl.kernel,
    out_type=(
        jax.ShapeDtypeStruct((NW, 4096), _i32),
        jax.ShapeDtypeStruct((L,), _i32),
    ),
    mesh=_mesh,
    compiler_params=_params,
    scratch_types=[
        pltpu.VMEM((2, WIN), _f32),
        pltpu.SemaphoreType.DMA,
        pltpu.SemaphoreType.DMA,
        pltpu.VMEM((2 * L * 2048,), _i32),
        pltpu.VMEM((8, 2048), _i32),
        pltpu.VMEM((2048,), _i32),
        pltpu.VMEM((4096,), _i32),
        pltpu.VMEM((L,), _i32),
    ],
)
def _kb(loss, ha, hout, stout, buf2, sem0, sem1, tbl, blockbuf, acc, hloc, stv):
    wid = _wid()
    base = wid * PER_W
    _reduce_rows(ha, blockbuf, acc, NW, 2048, 8)
    p1, cb1 = _select(acc, 0, 2048, K1)
    p2, cb2 = _select(acc, 0, 2048, K2)
    kr1 = K1 - cb1
    kr2 = K2 - cb2

    pref1 = plsc.bitcast(_ci(p1), _u32)
    pref2 = plsc.bitcast(_ci(p2), _u32)
    laneb = plsc.bitcast(lax.iota(_i32, L) * 2048, _u32)
    slot1 = _c32(L * 2048)
    zu = _c32(0)
    ones = jnp.ones((L,), _i32)
    _zero_words(tbl, 2 * L * 2048)

    def compute(par, c):

        @plsc.parallel_loop(0, VPW, unroll=16)
        def _(i):
            x = buf2[par, pl.ds(i * L, L)]
            key = _key_of(x)
            hi = key >> _c32(21)
            m1 = hi == pref1
            m2 = hi == pref2
            b = laneb | ((key >> _c32(10)) & _c32(0x7FF)) | jnp.where(m2, slot1, zu)
            plsc.addupdate_scatter(
                tbl, [plsc.bitcast(b, _i32)], ones,
                mask=jnp.logical_or(m1, m2))

        return c

    _stream(loss, buf2, (sem0, sem1), base, compute, 0)
    _lane_reduce(tbl, hloc, 2, 2048)
    pltpu.sync_copy(hloc, hout.at[wid])

    iota = lax.iota(_i32, L)
    st = (jnp.where(iota == 0, _ci(p1), _ci(0))
          + jnp.where(iota == 1, _ci(kr1), _ci(0))
          + jnp.where(iota == 2, _ci(p2), _ci(0))
          + jnp.where(iota == 3, _ci(kr2), _ci(0)))
    stv[...] = st

    @pl.when(wid == 0)
    def _():
        pltpu.sync_copy(stv, stout)


@functools.partial(
    pl.kernel,
    out_type=(
        jax.ShapeDtypeStruct((NW, 2048), _i32),
        jax.ShapeDtypeStruct((NW, 2048), _f32),
        jax.ShapeDtypeStruct((NW, 2 * L), _f32),
        jax.ShapeDtypeStruct((L,), _i32),
    ),
    mesh=_mesh,
    compiler_params=_params,
    scratch_types=[
        pltpu.VMEM((2, WIN), _f32),
        pltpu.SemaphoreType.DMA,
        pltpu.SemaphoreType.DMA,
        pltpu.VMEM((2 * L * 1024,), _i32),
        pltpu.VMEM((2 * L * 1024,), _f32),
        pltpu.VMEM((4, 4096), _i32),
        pltpu.VMEM((4096,), _i32),
        pltpu.VMEM((2048,), _i32),
        pltpu.VMEM((2048,), _f32),
        pltpu.VMEM((2 * L,), _f32),
        pltpu.VMEM((L,), _i32),
    ],
)
def _kc(loss, hb, stb, hout, sout, eout, stout,
        buf2, sem0, sem1, tbl, stbl, blockbuf, acc, hloc, sloc, erow, stv):
    wid = _wid()
    base = wid * PER_W
    pltpu.sync_copy(stb, stv)
    sv = stv[...]
    p1a = sv[0]
    kr1a = sv[1]
    p2a = sv[2]
    kr2a = sv[3]

    _reduce_rows(hb, blockbuf, acc, NW, 4096, 4)
    b1 = jnp.where(p1a == p2a, 2048, 0)
    p1b, cb1 = _select(acc, b1, 2048, kr1a)
    p2b, cb2 = _select(acc, 2048, 2048, kr2a)
    kr1 = kr1a - cb1
    kr2 = kr2a - cb2
    pref1 = ((plsc.bitcast(_ci(p1a), _u32) << _c32(11))
             | plsc.bitcast(_ci(p1b), _u32))
    pref2 = ((plsc.bitcast(_ci(p2a), _u32) << _c32(11))
             | plsc.bitcast(_ci(p2b), _u32))
    edge1 = pref1 << _c32(10)
    edge2 = pref2 << _c32(10)

    laneb = plsc.bitcast(lax.iota(_i32, L) * 1024, _u32)
    slot1 = _c32(L * 1024)
    zu = _c32(0)
    zf = jnp.zeros((L,), _f32)
    ones = jnp.ones((L,), _i32)
    _zero_words(tbl, 2 * L * 1024)
    _zero_words(stbl, 2 * L * 1024, _f32)

    def compute(par, carry):

        @plsc.parallel_loop(0, VPW, unroll=16, carry=carry)
        def inner(i, c):
            s1, s2 = c
            x = buf2[par, pl.ds(i * L, L)]
            key = _key_of(x)
            s1 = s1 + jnp.where(key < edge1, x, zf)
            s2 = s2 + jnp.where(key < edge2, x, zf)
            hi = key >> _c32(10)
            m1 = hi == pref1
            m2 = hi == pref2
            m = jnp.logical_or(m1, m2)
            idx = plsc.bitcast(
                laneb | (key & _c32(0x3FF)) | jnp.where(m2, slot1, zu), _i32)
            plsc.addupdate_scatter(tbl, [idx], ones, mask=m)
            plsc.addupdate_scatter(stbl, [idx], x, mask=m)
            return s1, s2

        return inner

    s1, s2 = _stream(loss, buf2, (sem0, sem1), base, compute, (zf, zf))

    _lane_reduce(tbl, hloc, 2, 1024)
    _lane_reduce(stbl, sloc, 2, 1024, _f32)
    pltpu.sync_copy(hloc, hout.at[wid])
    pltpu.sync_copy(sloc, sout.at[wid])
    erow[pl.ds(0, L)] = s1
    erow[pl.ds(L, L)] = s2
    pltpu.sync_copy(erow, eout.at[wid])

    pr1 = (p1a << 11) | p1b
    pr2 = (p2a << 11) | p2b
    iota = lax.iota(_i32, L)
    st = (jnp.where(iota == 0, _ci(pr1), _ci(0))
          + jnp.where(iota == 1, _ci(kr1), _ci(0))
          + jnp.where(iota == 2, _ci(pr2), _ci(0))
          + jnp.where(iota == 3, _ci(kr2), _ci(0)))
    stv[...] = st

    @pl.when(wid == 0)
    def _():
        pltpu.sync_copy(stv, stout)


@functools.partial(
    pl.kernel,
    out_type=jax.ShapeDtypeStruct((L,), _f32),
    mesh=_mesh,
    compiler_params=_params,
    scratch_types=[
        pltpu.VMEM((16, 2048), _i32),
        pltpu.VMEM((16, 2048), _f32),
        pltpu.VMEM((2048,), _i32),
        pltpu.VMEM((2048,), _f32),
        pltpu.VMEM((NW, 2 * L), _f32),
        pltpu.VMEM((L,), _i32),
        pltpu.VMEM((L,), _f32),
    ],
)
def _kd(hc, sc, ec, stc, rout, blockbuf, sblockbuf, acc, sacc, ebuf, stv, res):
    wid = _wid()

    @pl.when(wid == 0)
    def _():
        pltpu.sync_copy(stc, stv)
        sv = stv[...]
        pr1 = sv[0]
        kr1a = sv[1]
        pr2 = sv[2]
        kr2a = sv[3]

        _reduce_rows(hc, blockbuf, acc, NW, 2048, 16)
        _reduce_rows(sc, sblockbuf, sacc, NW, 2048, 16, _f32)

        zf = jnp.zeros((L,), _f32)
        pltpu.sync_copy(ec, ebuf)

        def erows(w, c):
            e1, e2 = c
            return e1 + ebuf[w, pl.ds(0, L)], e2 + ebuf[w, pl.ds(L, L)]

        e1, e2 = lax.fori_loop(0, NW, erows, (zf, zf))

        b1 = jnp.where(pr1 == pr2, 1024, 0)
        p1c, cb1, sb1 = _select(acc, b1, 1024, kr1a, sacc)
        p2c, cb2, sb2 = _select(acc, 1024, 1024, kr2a, sacc)
        kr1 = kr1a - cb1
        kr2 = kr2a - cb2
        t1 = ((plsc.bitcast(_ci(pr1), _u32) << _c32(10))
              | plsc.bitcast(_ci(p1c), _u32))
        t2 = ((plsc.bitcast(_ci(pr2), _u32) << _c32(10))
              | plsc.bitcast(_ci(p2c), _u32))

        def decode(t):
            pos = t >= _c32(0x80000000)
            bits = t ^ jnp.where(pos, _c32(0x80000000), _c32(0xFFFFFFFF))
            return plsc.bitcast(bits, _f32)

        iota = lax.iota(_i32, L)
        lane0 = iota == 0
        corr1 = jnp.where(lane0, _ci(kr1).astype(_f32) * decode(t1), zf)
        corr2 = jnp.where(lane0, _ci(kr2).astype(_f32) * decode(t2), zf)
        dvec = (e2 - e1) + (sb2 - sb1) + (corr2 - corr1)
        res[...] = plsc.cumsum(dvec) / _cf(float(CNT))
        pltpu.sync_copy(res, rout)


def kernel(loss):
    assert loss.shape == (N,) and loss.dtype == jnp.float32
    ha = _ka(loss)
    hb, stb = _kb(loss, ha)
    hc, sc, ec, stc = _kc(loss, hb, stb)
    out = _kd(hc, sc, ec, stc)
    return out[L - 1]
